# Initial kernel scaffold; baseline (speedup 1.0000x reference)
import jax
import jax.numpy as jnp
from jax.experimental import pallas as pl


def kernel(inputs, tables, W, b):
    raise NotImplementedError("write your pallas kernel here")


# speedup vs baseline: 20.1392x; 20.1392x over previous
import functools

import jax
import jax.numpy as jnp
import numpy as np
from jax import lax
from jax.experimental import pallas as pl
from jax.experimental.pallas import tpu as pltpu
from jax.experimental.pallas import tpu_sc as plsc

L = 12
F = 2
T = 2 ** 19
BASE = 16
FINEST = 2048
GROWTH = float(np.exp((np.log(FINEST) - np.log(BASE)) / (L - 1)))
RES = [float(np.floor(BASE * (GROWTH ** l))) for l in range(L)]
P1 = np.uint32(2654435761)
P2 = np.uint32(805459861)

N_PTS = 262144
D_OUT = 65

NC = 2
NS = 16
NW = NC * NS
LANES = 16

CH = 256
PTS_PER_W = N_PTS // NW
NCHUNK = PTS_PER_W // CH
G = NW * NCHUNK
NIDX = 8 * CH
WIN = 128
NWIN = NIDX // WIN
NBLK = L * T // 8


def _sc_embed(pts, tables_blk, resv, lofsv):
    mesh = plsc.VectorSubcoreMesh(core_axis_name="c", subcore_axis_name="s")
    cp = pltpu.CompilerParams(
        needs_layout_passes=False, use_tc_tiling_on_sc=False)

    @functools.partial(
        pl.kernel,
        mesh=mesh,
        compiler_params=cp,
        out_type=jax.ShapeDtypeStruct((G, 2 * L, CH), jnp.float32),
        scratch_types=[
            pltpu.VMEM((3, CH), jnp.float32),
            pltpu.VMEM((NWIN, WIN), jnp.int32),
            pltpu.VMEM((NIDX,), jnp.int32),
            pltpu.VMEM((NIDX,), jnp.float32),
            pltpu.VMEM((NIDX, 16), jnp.float32),
            pltpu.VMEM((2 * L, CH), jnp.float32),
            pltpu.VMEM((L, LANES), jnp.float32),
            pltpu.VMEM((L, LANES), jnp.int32),
            pltpu.SemaphoreType.DMA,
            pltpu.SemaphoreType.DMA,
        ],
    )
    def body(pts_hbm, tab_hbm, res_hbm, lofs_hbm, feats_hbm,
             x_v, idx_v, o_v, w_v, rows_v, f_v, res_v, lofs_v, gsem, dsem):
        wid = lax.axis_index("s") * NC + lax.axis_index("c")
        pltpu.sync_copy(res_hbm, res_v)
        pltpu.sync_copy(lofs_hbm, lofs_v)
        iota = lax.broadcasted_iota(jnp.int32, (LANES,), 0)

        @pl.loop(0, NCHUNK)
        def _chunk(ch):
            g = wid * NCHUNK + ch
            pltpu.async_copy(pts_hbm.at[g], x_v, dsem).wait()

            @pl.loop(0, L)
            def _level(l):
                res = res_v[l]
                lofs = lofs_v[l]

                @pl.loop(0, CH, step=LANES)
                def _compute(p):
                    x0 = x_v[0, pl.ds(p, LANES)]
                    x1 = x_v[1, pl.ds(p, LANES)]
                    x2 = x_v[2, pl.ds(p, LANES)]
                    pos0 = x0 * res
                    pos1 = x1 * res
                    pos2 = x2 * res
                    i0 = pos0.astype(jnp.int32)
                    i1 = pos1.astype(jnp.int32)
                    i2 = pos2.astype(jnp.int32)
                    f0 = pos0 - i0.astype(jnp.float32)
                    f1 = pos1 - i1.astype(jnp.float32)
                    f2 = pos2 - i2.astype(jnp.float32)
                    g0 = 1.0 - f0
                    g1 = 1.0 - f1
                    g2 = 1.0 - f2
                    c0 = i0.astype(jnp.uint32)
                    a1 = i1.astype(jnp.uint32) * P1
                    a2 = i2.astype(jnp.uint32) * P2
                    xs = (c0, c0 + jnp.uint32(1))
                    ys = (a1, a1 + P1)
                    zs = (a2, a2 + P2)
                    wx = (g0, f0)
                    wy = (g1, f1)
                    wz = (g2, f2)
                    mask = jnp.uint32(T - 1)
                    seven = jnp.uint32(7)
                    for i in range(2):
                        for j in range(2):
                            wxy = wx[i] * wy[j]
                            hxy = xs[i] ^ ys[j]
                            for k in range(2):
                                ci = i * 4 + j * 2 + k
                                h = (hxy ^ zs[k]) & mask
                                blk = (h >> 3).astype(jnp.int32) + lofs
                                col = ((h & seven) << 1).astype(jnp.int32)
                                kofs = ci * CH + p
                                idx_v[kofs // WIN,
                                      pl.ds(kofs % WIN, LANES)] = blk
                                o_v[pl.ds(kofs, LANES)] = col
                                w_v[pl.ds(kofs, LANES)] = wxy * wz[k]

                copies = [
                    pltpu.async_copy(
                        tab_hbm.at[idx_v.at[j]],
                        rows_v.at[pl.ds(j * WIN, WIN)],
                        gsem,
                    )
                    for j in range(NWIN)
                ]
                for cp_ in copies:
                    cp_.wait()

                @pl.loop(0, CH, step=LANES)
                def _accum(p):
                    acc0 = jnp.zeros((LANES,), jnp.float32)
                    acc1 = jnp.zeros((LANES,), jnp.float32)
                    for ci in range(8):
                        kofs = ci * CH + p
                        wv = w_v[pl.ds(kofs, LANES)]
                        col = o_v[pl.ds(kofs, LANES)]
                        rid = iota + kofs
                        r0 = plsc.load_gather(rows_v, [rid, col])
                        r1 = plsc.load_gather(rows_v, [rid, col + 1])
                        acc0 = acc0 + wv * r0
                        acc1 = acc1 + wv * r1
                    f_v[2 * l, pl.ds(p, LANES)] = acc0
                    f_v[2 * l + 1, pl.ds(p, LANES)] = acc1

            pltpu.async_copy(f_v, feats_hbm.at[g], dsem).wait()

    return body(pts, tables_blk, resv, lofsv)


def _mm_body(f_ref, w_ref, b_ref, o_ref):
    f = f_ref[0]
    o_ref[...] = (
        lax.dot_general(
            f, w_ref[...],
            dimension_numbers=(((0,), (1,)), ((), ())),
            preferred_element_type=jnp.float32,
            precision=lax.Precision.HIGHEST,
        )
        + b_ref[...]
    )


def _tc_linear(feats, W, b):
    return pl.pallas_call(
        _mm_body,
        grid=(G,),
        in_specs=[
            pl.BlockSpec((1, 2 * L, CH), lambda g: (g, 0, 0)),
            pl.BlockSpec((D_OUT, 2 * L), lambda g: (0, 0)),
            pl.BlockSpec((1, D_OUT), lambda g: (0, 0)),
        ],
        out_specs=pl.BlockSpec((CH, D_OUT), lambda g: (g, 0)),
        out_shape=jax.ShapeDtypeStruct((N_PTS, D_OUT), jnp.float32),
    )(feats, W, b)


def kernel(inputs, tables, W, b):
    pts = inputs.T.reshape(3, G, CH).transpose(1, 0, 2)
    tables_blk = tables.reshape(NBLK, 16)
    resv = jnp.tile(jnp.asarray(RES, jnp.float32)[:, None], (1, LANES))
    lofsv = jnp.tile(
        (jnp.arange(L, dtype=jnp.int32) * (T // 8))[:, None], (1, LANES))
    feats = _sc_embed(pts, tables_blk, resv, lofsv)
    return _tc_linear(feats, W, b.reshape(1, D_OUT))
